# Initial kernel scaffold; baseline (speedup 1.0000x reference)
import jax
import jax.numpy as jnp
from jax.experimental import pallas as pl


def kernel(x, gamma, beta):
    raise NotImplementedError("write your pallas kernel here")


# speedup vs baseline: 1.5817x; 1.5817x over previous
import functools

import jax
import jax.numpy as jnp
from jax.experimental import pallas as pl
from jax.experimental.pallas import tpu as pltpu

_EPS = 1e-5


def _bn_strip_kernel(x_ref, g_ref, b_ref, y_ref, *, inv_n, eps):
    x = x_ref[...].astype(jnp.float32)
    inv = jnp.float32(inv_n)
    m1 = jnp.sum(x, axis=0, keepdims=True) * inv
    m2 = jnp.sum(x * x, axis=0, keepdims=True) * inv
    var = jnp.maximum(m2 - m1 * m1, 0.0)
    k = g_ref[...] * jax.lax.rsqrt(var + eps)
    y_ref[...] = ((x - m1) * k + b_ref[...]).astype(y_ref.dtype)


def _strip_width(n, c, itemsize):
    if c % 128 != 0:
        return c
    w = 128
    while w < c and n * 2 * w * (2 * itemsize + 2 * itemsize + 8) > 56 * 1024 * 1024:
        break
    return w


def kernel(x, gamma, beta):
    n, c = x.shape
    g2d = gamma.astype(jnp.float32).reshape(1, c)
    b2d = beta.astype(jnp.float32).reshape(1, c)

    tile_c = _strip_width(n, c, x.dtype.itemsize)
    num_strips = c // tile_c

    body = functools.partial(_bn_strip_kernel, inv_n=1.0 / n, eps=_EPS)
    return pl.pallas_call(
        body,
        out_shape=jax.ShapeDtypeStruct((n, c), x.dtype),
        grid=(num_strips,),
        in_specs=[
            pl.BlockSpec((n, tile_c), lambda j: (0, j)),
            pl.BlockSpec((1, tile_c), lambda j: (0, j)),
            pl.BlockSpec((1, tile_c), lambda j: (0, j)),
        ],
        out_specs=pl.BlockSpec((n, tile_c), lambda j: (0, j)),
        compiler_params=pltpu.CompilerParams(
            dimension_semantics=("parallel",),
            vmem_limit_bytes=56 * 1024 * 1024,
        ),
    )(x, g2d, b2d)
